# baseline (speedup 1.0000x reference)
import functools

import jax
import jax.numpy as jnp
import numpy as np
from jax.experimental import pallas as pl
from jax.experimental.pallas import tpu as pltpu

S, D = 2048, 1024
H, KVH, HD = 16, 8, 64
HHD = HD // 2
E, FF = 8, 2048
THETA = 10000.0
EPS = 1e-5
RB = 256
NRB = S // RB

_INTERPRET = False


def _ln(x, w, b):
    mu = jnp.mean(x, axis=-1, keepdims=True)
    xc = x - mu
    var = jnp.mean(xc * xc, axis=-1, keepdims=True)
    return xc * jax.lax.rsqrt(var + EPS) * w + b


def _qkv_body(x_ref, cos_ref, sin_ref, ln1w_ref, ln1b_ref, wq_ref, bq_ref,
              wk_ref, bk_ref, wv_ref, bv_ref, q_ref, k_ref, v_ref):
    x = x_ref[...]
    h = _ln(x, ln1w_ref[...], ln1b_ref[...]).astype(jnp.bfloat16)
    q = jnp.dot(h, wq_ref[...], preferred_element_type=jnp.float32) + bq_ref[...]
    k = jnp.dot(h, wk_ref[...], preferred_element_type=jnp.float32) + bk_ref[...]
    v = jnp.dot(h, wv_ref[...], preferred_element_type=jnp.float32) + bv_ref[...]
    cos = cos_ref[...]
    sin = sin_ref[...]
    cq = jnp.concatenate([cos] * H, axis=1)
    sq = jnp.concatenate([sin] * H, axis=1)
    ck = jnp.concatenate([cos] * KVH, axis=1)
    sk = jnp.concatenate([sin] * KVH, axis=1)
    q1, q2 = q[:, :H * HHD], q[:, H * HHD:]
    k1, k2 = k[:, :KVH * HHD], k[:, KVH * HHD:]
    q_ref[...] = jnp.concatenate(
        [q1 * cq - q2 * sq, q2 * cq + q1 * sq], axis=1).astype(jnp.bfloat16)
    k_ref[...] = jnp.concatenate(
        [k1 * ck - k2 * sk, k2 * ck + k1 * sk], axis=1).astype(jnp.bfloat16)
    v_ref[...] = v.astype(jnp.bfloat16)


def _attn_body(q_ref, k_ref, v_ref, o_ref):
    qi = pl.program_id(0)
    q = q_ref[...]
    rows = qi * RB + jax.lax.broadcasted_iota(jnp.int32, (RB, RB), 0)
    colw = jax.lax.broadcasted_iota(jnp.int32, (RB, RB), 1)
    for h in range(H):
        kv = h // (H // KVH)
        q1 = q[:, h * HHD:(h + 1) * HHD]
        q2 = q[:, H * HHD + h * HHD: H * HHD + (h + 1) * HHD]

        def step(ki, carry):
            m, l, acc = carry
            krows = pl.ds(ki * RB, RB)
            k1 = k_ref[krows, kv * HHD:(kv + 1) * HHD]
            k2 = k_ref[krows, KVH * HHD + kv * HHD: KVH * HHD + (kv + 1) * HHD]
            vc = v_ref[krows, kv * HD:(kv + 1) * HD]
            s = jax.lax.dot_general(q1, k1, (((1,), (1,)), ((), ())),
                                    preferred_element_type=jnp.float32)
            s = s + jax.lax.dot_general(q2, k2, (((1,), (1,)), ((), ())),
                                        preferred_element_type=jnp.float32)
            s = jnp.where(ki * RB + colw <= rows, s * 0.125, jnp.float32(-1e30))
            mnew = jnp.maximum(m, jnp.max(s, axis=1, keepdims=True))
            corr = jnp.exp(m - mnew)
            p = jnp.exp(s - mnew)
            l2 = l * corr + jnp.sum(p, axis=1, keepdims=True)
            acc2 = acc * corr + jnp.dot(p.astype(jnp.bfloat16), vc,
                                        preferred_element_type=jnp.float32)
            return mnew, l2, acc2

        m0 = jnp.full((RB, 1), -1e30, jnp.float32)
        l0 = jnp.zeros((RB, 1), jnp.float32)
        acc0 = jnp.zeros((RB, HD), jnp.float32)
        m, l, acc = jax.lax.fori_loop(0, qi + 1, step, (m0, l0, acc0))
        o_ref[:, h * HD:(h + 1) * HD] = (acc / l).astype(jnp.bfloat16)


def _post_body(x_ref, attn_ref, wo_ref, bo_ref, ln2w_ref, ln2b_ref, wr_ref,
               x2_ref, h2_ref, g_ref):
    x2 = x_ref[...] + jnp.dot(attn_ref[...], wo_ref[...],
                              preferred_element_type=jnp.float32) + bo_ref[...]
    h2 = _ln(x2, ln2w_ref[...], ln2b_ref[...])
    logits128 = jnp.dot(h2, wr_ref[...], preferred_element_type=jnp.float32)
    logits = logits128[:, :E]
    m = jnp.max(logits, axis=1, keepdims=True)
    p = jnp.exp(logits - m)
    rp = p / jnp.sum(p, axis=1, keepdims=True)
    iota = jax.lax.broadcasted_iota(jnp.int32, (RB, E), 1)
    v1 = jnp.max(rp, axis=1, keepdims=True)
    i1 = jnp.min(jnp.where(rp == v1, iota, E), axis=1, keepdims=True)
    rp2 = jnp.where(iota == i1, -1.0, rp)
    v2 = jnp.max(rp2, axis=1, keepdims=True)
    i2 = jnp.min(jnp.where(rp2 == v2, iota, E), axis=1, keepdims=True)
    denom = v1 + v2
    iota128 = jax.lax.broadcasted_iota(jnp.int32, (RB, 128), 1)
    g = (jnp.where(iota128 == i1, v1, 0.0)
         + jnp.where(iota128 == i2, v2, 0.0)) / denom
    x2_ref[...] = x2
    h2_ref[...] = h2
    g_ref[...] = g


def _moe_body(h2_ref, g_ref, x2_ref, w1_ref, w3_ref, w2_ref, out_ref, acc_ref):
    e = pl.program_id(0)
    si = pl.program_id(1)
    rows = pl.ds(si * RB, RB)
    h2 = h2_ref[...].astype(jnp.bfloat16)
    a = jnp.dot(h2, w1_ref[0], preferred_element_type=jnp.float32)
    b = jnp.dot(h2, w3_ref[0], preferred_element_type=jnp.float32)
    act = (a * jax.nn.sigmoid(a) * b).astype(jnp.bfloat16)
    y = jnp.dot(act, w2_ref[0], preferred_element_type=jnp.float32)
    iota128 = jax.lax.broadcasted_iota(jnp.int32, (RB, 128), 1)
    ge = jnp.sum(jnp.where(iota128 == e, g_ref[...], 0.0), axis=1, keepdims=True)
    contrib = ge * y

    @pl.when(e == 0)
    def _():
        acc_ref[rows, :] = contrib

    @pl.when(e > 0)
    def _():
        acc_ref[rows, :] = acc_ref[rows, :] + contrib

    @pl.when(e == E - 1)
    def _():
        out_ref[...] = acc_ref[rows, :] + x2_ref[...]


def _split_halves(w):
    g = w.shape[-1] // HD
    w3d = w.reshape(*w.shape[:-1], g, HD)
    return jnp.concatenate(
        [w3d[..., :HHD].reshape(*w.shape[:-1], g * HHD),
         w3d[..., HHD:].reshape(*w.shape[:-1], g * HHD)], axis=-1)


def kernel(hidden_states, attention_mask, position_ids, ln1_w, ln1_b, wq, bq,
           wk, bk, wv, bv, wo, bo, ln2_w, ln2_b, w_router, w1, w3, w2):
    x = hidden_states.reshape(S, D)
    pos = position_ids.reshape(S).astype(jnp.float32)
    inv = jnp.asarray(1.0 / (THETA ** (np.arange(0, HD, 2) / HD)), jnp.float32)
    ang = pos[:, None] * inv[None, :]
    cos = jnp.cos(ang)
    sin = jnp.sin(ang)

    wq_p = _split_halves(wq)
    bq_p = _split_halves(bq[None, :])[0]
    wk_p = _split_halves(wk)
    bk_p = _split_halves(bk[None, :])[0]
    wr_pad = jnp.pad(w_router, ((0, 0), (0, 128 - E)))

    row_spec = pl.BlockSpec((RB, D), lambda i: (i, 0))
    full = lambda *shape: pl.BlockSpec(shape, lambda *i: (0,) * len(shape))

    qf, kf, vf = pl.pallas_call(
        _qkv_body,
        grid=(NRB,),
        in_specs=[
            row_spec,
            pl.BlockSpec((RB, HHD), lambda i: (i, 0)),
            pl.BlockSpec((RB, HHD), lambda i: (i, 0)),
            full(D), full(D),
            full(D, H * HD), full(H * HD),
            full(D, KVH * HD), full(KVH * HD),
            full(D, KVH * HD), full(KVH * HD),
        ],
        out_specs=[
            pl.BlockSpec((RB, H * HD), lambda i: (i, 0)),
            pl.BlockSpec((RB, KVH * HD), lambda i: (i, 0)),
            pl.BlockSpec((RB, KVH * HD), lambda i: (i, 0)),
        ],
        out_shape=[
            jax.ShapeDtypeStruct((S, H * HD), jnp.bfloat16),
            jax.ShapeDtypeStruct((S, KVH * HD), jnp.bfloat16),
            jax.ShapeDtypeStruct((S, KVH * HD), jnp.bfloat16),
        ],
        interpret=_INTERPRET,
    )(x, cos, sin, ln1_w, ln1_b, wq_p.astype(jnp.bfloat16),
      bq_p, wk_p.astype(jnp.bfloat16), bk_p, wv.astype(jnp.bfloat16), bv)

    attn = pl.pallas_call(
        _attn_body,
        grid=(NRB,),
        in_specs=[
            pl.BlockSpec((RB, H * HD), lambda i: (i, 0)),
            full(S, KVH * HD),
            full(S, KVH * HD),
        ],
        out_specs=pl.BlockSpec((RB, H * HD), lambda i: (i, 0)),
        out_shape=jax.ShapeDtypeStruct((S, H * HD), jnp.bfloat16),
        interpret=_INTERPRET,
    )(qf, kf, vf)

    x2, h2, gates = pl.pallas_call(
        _post_body,
        grid=(NRB,),
        in_specs=[
            row_spec,
            pl.BlockSpec((RB, H * HD), lambda i: (i, 0)),
            full(H * HD, D), full(D),
            full(D), full(D),
            full(D, 128),
        ],
        out_specs=[row_spec, row_spec, pl.BlockSpec((RB, 128), lambda i: (i, 0))],
        out_shape=[
            jax.ShapeDtypeStruct((S, D), jnp.float32),
            jax.ShapeDtypeStruct((S, D), jnp.float32),
            jax.ShapeDtypeStruct((S, 128), jnp.float32),
        ],
        interpret=_INTERPRET,
    )(x, attn, wo.astype(jnp.bfloat16), bo, ln2_w, ln2_b, wr_pad)

    w1b = w1.astype(jnp.bfloat16)
    w3b = w3.astype(jnp.bfloat16)
    w2b = w2.astype(jnp.bfloat16)

    out = pl.pallas_call(
        _moe_body,
        grid=(E, NRB),
        in_specs=[
            pl.BlockSpec((RB, D), lambda e, s: (s, 0)),
            pl.BlockSpec((RB, 128), lambda e, s: (s, 0)),
            pl.BlockSpec((RB, D), lambda e, s: (s, 0)),
            pl.BlockSpec((1, D, FF), lambda e, s: (e, 0, 0)),
            pl.BlockSpec((1, D, FF), lambda e, s: (e, 0, 0)),
            pl.BlockSpec((1, FF, D), lambda e, s: (e, 0, 0)),
        ],
        out_specs=pl.BlockSpec(
            (RB, D), lambda e, s: (jnp.where(e == E - 1, s, 0), 0)),
        out_shape=jax.ShapeDtypeStruct((S, D), jnp.float32),
        scratch_shapes=[pltpu.VMEM((S, D), jnp.float32)],
        interpret=_INTERPRET,
    )(h2, gates, x2, w1b, w3b, w2b)

    return out.reshape(1, S, D)


# speedup vs baseline: 1.0679x; 1.0679x over previous
import functools

import jax
import jax.numpy as jnp
import numpy as np
from jax.experimental import pallas as pl
from jax.experimental.pallas import tpu as pltpu

S, D = 2048, 1024
H, KVH, HD = 16, 8, 64
HHD = HD // 2
E, FF = 8, 2048
THETA = 10000.0
EPS = 1e-5
RB = 256
NRB = S // RB

_INTERPRET = False


def _ln(x, w, b):
    mu = jnp.mean(x, axis=-1, keepdims=True)
    xc = x - mu
    var = jnp.mean(xc * xc, axis=-1, keepdims=True)
    return xc * jax.lax.rsqrt(var + EPS) * w + b


def _qkv_body(x_ref, cos_ref, sin_ref, ln1w_ref, ln1b_ref, wq_ref, bq_ref,
              wk_ref, bk_ref, wv_ref, bv_ref, q_ref, k_ref, v_ref):
    x = x_ref[...]
    h = _ln(x, ln1w_ref[...], ln1b_ref[...]).astype(jnp.bfloat16)
    q = jnp.dot(h, wq_ref[...], preferred_element_type=jnp.float32) + bq_ref[...]
    k = jnp.dot(h, wk_ref[...], preferred_element_type=jnp.float32) + bk_ref[...]
    v = jnp.dot(h, wv_ref[...], preferred_element_type=jnp.float32) + bv_ref[...]
    cos = cos_ref[...]
    sin = sin_ref[...]
    cq = jnp.concatenate([cos] * H, axis=1)
    sq = jnp.concatenate([sin] * H, axis=1)
    ck = jnp.concatenate([cos] * KVH, axis=1)
    sk = jnp.concatenate([sin] * KVH, axis=1)
    q1, q2 = q[:, :H * HHD], q[:, H * HHD:]
    k1, k2 = k[:, :KVH * HHD], k[:, KVH * HHD:]
    q_ref[...] = jnp.concatenate(
        [q1 * cq - q2 * sq, q2 * cq + q1 * sq], axis=1).astype(jnp.bfloat16)
    k_ref[...] = jnp.concatenate(
        [k1 * ck - k2 * sk, k2 * ck + k1 * sk], axis=1).astype(jnp.bfloat16)
    v_ref[...] = v.astype(jnp.bfloat16)


def _attn_body(q_ref, k_ref, v_ref, o_ref, m_scr, l_scr, acc_scr):
    qi = pl.program_id(0)
    ki = pl.program_id(1)

    @pl.when(ki == 0)
    def _():
        m_scr[...] = jnp.full((RB, H), -1e30, jnp.float32)
        l_scr[...] = jnp.zeros((RB, H), jnp.float32)
        acc_scr[...] = jnp.zeros((RB, H * HD), jnp.float32)

    @pl.when(ki <= qi)
    def _():
        q = q_ref[...]
        roww = jax.lax.broadcasted_iota(jnp.int32, (RB, RB), 0)
        colw = jax.lax.broadcasted_iota(jnp.int32, (RB, RB), 1)
        mask = (ki * RB + colw) <= (qi * RB + roww)
        for h in range(H):
            kv = h // (H // KVH)
            q1 = q[:, h * HHD:(h + 1) * HHD]
            q2 = q[:, H * HHD + h * HHD: H * HHD + (h + 1) * HHD]
            k1 = k_ref[:, kv * HHD:(kv + 1) * HHD]
            k2 = k_ref[:, KVH * HHD + kv * HHD: KVH * HHD + (kv + 1) * HHD]
            vc = v_ref[:, kv * HD:(kv + 1) * HD]
            s = jax.lax.dot_general(q1, k1, (((1,), (1,)), ((), ())),
                                    preferred_element_type=jnp.float32)
            s = s + jax.lax.dot_general(q2, k2, (((1,), (1,)), ((), ())),
                                        preferred_element_type=jnp.float32)
            s = jnp.where(mask, s * 0.125, jnp.float32(-1e30))
            m = m_scr[:, h:h + 1]
            mnew = jnp.maximum(m, jnp.max(s, axis=1, keepdims=True))
            corr = jnp.exp(m - mnew)
            p = jnp.exp(s - mnew)
            m_scr[:, h:h + 1] = mnew
            l_scr[:, h:h + 1] = l_scr[:, h:h + 1] * corr + \
                jnp.sum(p, axis=1, keepdims=True)
            acch = acc_scr[:, h * HD:(h + 1) * HD]
            acc_scr[:, h * HD:(h + 1) * HD] = acch * corr + jnp.dot(
                p.astype(jnp.bfloat16), vc, preferred_element_type=jnp.float32)

    @pl.when(ki == qi)
    def _():
        for h in range(H):
            o_ref[:, h * HD:(h + 1) * HD] = (
                acc_scr[:, h * HD:(h + 1) * HD] / l_scr[:, h:h + 1]
            ).astype(jnp.bfloat16)


def _post_body(x_ref, attn_ref, wo_ref, bo_ref, ln2w_ref, ln2b_ref, wr_ref,
               x2_ref, h2_ref, g_ref):
    x2 = x_ref[...] + jnp.dot(attn_ref[...], wo_ref[...],
                              preferred_element_type=jnp.float32) + bo_ref[...]
    h2 = _ln(x2, ln2w_ref[...], ln2b_ref[...])
    logits128 = jnp.dot(h2, wr_ref[...], preferred_element_type=jnp.float32)
    logits = logits128[:, :E]
    m = jnp.max(logits, axis=1, keepdims=True)
    p = jnp.exp(logits - m)
    rp = p / jnp.sum(p, axis=1, keepdims=True)
    iota = jax.lax.broadcasted_iota(jnp.int32, (RB, E), 1)
    v1 = jnp.max(rp, axis=1, keepdims=True)
    i1 = jnp.min(jnp.where(rp == v1, iota, E), axis=1, keepdims=True)
    rp2 = jnp.where(iota == i1, -1.0, rp)
    v2 = jnp.max(rp2, axis=1, keepdims=True)
    i2 = jnp.min(jnp.where(rp2 == v2, iota, E), axis=1, keepdims=True)
    denom = v1 + v2
    iota128 = jax.lax.broadcasted_iota(jnp.int32, (RB, 128), 1)
    g = (jnp.where(iota128 == i1, v1, 0.0)
         + jnp.where(iota128 == i2, v2, 0.0)) / denom
    x2_ref[...] = x2
    h2_ref[...] = h2
    g_ref[...] = g


def _moe_body(h2_ref, g_ref, x2_ref, w1_ref, w3_ref, w2_ref, out_ref, acc_ref):
    e = pl.program_id(0)
    si = pl.program_id(1)
    rows = pl.ds(si * RB, RB)
    h2 = h2_ref[...].astype(jnp.bfloat16)
    a = jnp.dot(h2, w1_ref[0], preferred_element_type=jnp.float32)
    b = jnp.dot(h2, w3_ref[0], preferred_element_type=jnp.float32)
    act = (a * jax.nn.sigmoid(a) * b).astype(jnp.bfloat16)
    y = jnp.dot(act, w2_ref[0], preferred_element_type=jnp.float32)
    iota128 = jax.lax.broadcasted_iota(jnp.int32, (RB, 128), 1)
    ge = jnp.sum(jnp.where(iota128 == e, g_ref[...], 0.0), axis=1, keepdims=True)
    contrib = ge * y

    @pl.when(e == 0)
    def _():
        acc_ref[rows, :] = contrib

    @pl.when(e > 0)
    def _():
        acc_ref[rows, :] = acc_ref[rows, :] + contrib

    @pl.when(e == E - 1)
    def _():
        out_ref[...] = acc_ref[rows, :] + x2_ref[...]


def _split_halves(w):
    g = w.shape[-1] // HD
    w3d = w.reshape(*w.shape[:-1], g, HD)
    return jnp.concatenate(
        [w3d[..., :HHD].reshape(*w.shape[:-1], g * HHD),
         w3d[..., HHD:].reshape(*w.shape[:-1], g * HHD)], axis=-1)


def kernel(hidden_states, attention_mask, position_ids, ln1_w, ln1_b, wq, bq,
           wk, bk, wv, bv, wo, bo, ln2_w, ln2_b, w_router, w1, w3, w2):
    x = hidden_states.reshape(S, D)
    pos = position_ids.reshape(S).astype(jnp.float32)
    inv = jnp.asarray(1.0 / (THETA ** (np.arange(0, HD, 2) / HD)), jnp.float32)
    ang = pos[:, None] * inv[None, :]
    cos = jnp.cos(ang)
    sin = jnp.sin(ang)

    wq_p = _split_halves(wq)
    bq_p = _split_halves(bq[None, :])[0]
    wk_p = _split_halves(wk)
    bk_p = _split_halves(bk[None, :])[0]
    wr_pad = jnp.pad(w_router, ((0, 0), (0, 128 - E)))

    row_spec = pl.BlockSpec((RB, D), lambda i: (i, 0))
    full = lambda *shape: pl.BlockSpec(shape, lambda *i: (0,) * len(shape))

    qf, kf, vf = pl.pallas_call(
        _qkv_body,
        grid=(NRB,),
        in_specs=[
            row_spec,
            pl.BlockSpec((RB, HHD), lambda i: (i, 0)),
            pl.BlockSpec((RB, HHD), lambda i: (i, 0)),
            full(D), full(D),
            full(D, H * HD), full(H * HD),
            full(D, KVH * HD), full(KVH * HD),
            full(D, KVH * HD), full(KVH * HD),
        ],
        out_specs=[
            pl.BlockSpec((RB, H * HD), lambda i: (i, 0)),
            pl.BlockSpec((RB, KVH * HD), lambda i: (i, 0)),
            pl.BlockSpec((RB, KVH * HD), lambda i: (i, 0)),
        ],
        out_shape=[
            jax.ShapeDtypeStruct((S, H * HD), jnp.bfloat16),
            jax.ShapeDtypeStruct((S, KVH * HD), jnp.bfloat16),
            jax.ShapeDtypeStruct((S, KVH * HD), jnp.bfloat16),
        ],
        interpret=_INTERPRET,
    )(x, cos, sin, ln1_w, ln1_b, wq_p.astype(jnp.bfloat16),
      bq_p, wk_p.astype(jnp.bfloat16), bk_p, wv.astype(jnp.bfloat16), bv)

    attn = pl.pallas_call(
        _attn_body,
        grid=(NRB, NRB),
        in_specs=[
            pl.BlockSpec((RB, H * HD), lambda i, j: (i, 0)),
            pl.BlockSpec((RB, KVH * HD), lambda i, j: (jnp.minimum(i, j), 0)),
            pl.BlockSpec((RB, KVH * HD), lambda i, j: (jnp.minimum(i, j), 0)),
        ],
        out_specs=pl.BlockSpec((RB, H * HD), lambda i, j: (i, 0)),
        out_shape=jax.ShapeDtypeStruct((S, H * HD), jnp.bfloat16),
        scratch_shapes=[
            pltpu.VMEM((RB, H), jnp.float32),
            pltpu.VMEM((RB, H), jnp.float32),
            pltpu.VMEM((RB, H * HD), jnp.float32),
        ],
        interpret=_INTERPRET,
    )(qf, kf, vf)

    x2, h2, gates = pl.pallas_call(
        _post_body,
        grid=(NRB,),
        in_specs=[
            row_spec,
            pl.BlockSpec((RB, H * HD), lambda i: (i, 0)),
            full(H * HD, D), full(D),
            full(D), full(D),
            full(D, 128),
        ],
        out_specs=[row_spec, row_spec, pl.BlockSpec((RB, 128), lambda i: (i, 0))],
        out_shape=[
            jax.ShapeDtypeStruct((S, D), jnp.float32),
            jax.ShapeDtypeStruct((S, D), jnp.float32),
            jax.ShapeDtypeStruct((S, 128), jnp.float32),
        ],
        interpret=_INTERPRET,
    )(x, attn, wo.astype(jnp.bfloat16), bo, ln2_w, ln2_b, wr_pad)

    w1b = w1.astype(jnp.bfloat16)
    w3b = w3.astype(jnp.bfloat16)
    w2b = w2.astype(jnp.bfloat16)

    out = pl.pallas_call(
        _moe_body,
        grid=(E, NRB),
        in_specs=[
            pl.BlockSpec((RB, D), lambda e, s: (s, 0)),
            pl.BlockSpec((RB, 128), lambda e, s: (s, 0)),
            pl.BlockSpec((RB, D), lambda e, s: (s, 0)),
            pl.BlockSpec((1, D, FF), lambda e, s: (e, 0, 0)),
            pl.BlockSpec((1, D, FF), lambda e, s: (e, 0, 0)),
            pl.BlockSpec((1, FF, D), lambda e, s: (e, 0, 0)),
        ],
        out_specs=pl.BlockSpec(
            (RB, D), lambda e, s: (jnp.where(e == E - 1, s, 0), 0)),
        out_shape=jax.ShapeDtypeStruct((S, D), jnp.float32),
        scratch_shapes=[pltpu.VMEM((S, D), jnp.float32)],
        interpret=_INTERPRET,
    )(h2, gates, x2, w1b, w3b, w2b)

    return out.reshape(1, S, D)


# speedup vs baseline: 1.4937x; 1.3987x over previous
import functools

import jax
import jax.numpy as jnp
import numpy as np
from jax import lax
from jax.experimental import pallas as pl
from jax.experimental.pallas import tpu as pltpu
from jax.experimental.pallas import tpu_sc as plsc

S, D = 2048, 1024
H, KVH, HD = 16, 8, 64
HHD = HD // 2
E, FF = 8, 2048
THETA = 10000.0
EPS = 1e-5
RB = 256
NRB = S // RB

A = 2 * S
BLK = 256
PADMAX = A + E * BLK
NBLK = PADMAX // BLK
BUFROWS = A + PADMAX
NW = 32
PB = PADMAX // NW
CH = 64

_INTERPRET = False


def _ln(x, w, b):
    mu = jnp.mean(x, axis=-1, keepdims=True)
    xc = x - mu
    var = jnp.mean(xc * xc, axis=-1, keepdims=True)
    return xc * jax.lax.rsqrt(var + EPS) * w + b


def _qkv_body(x_ref, cos_ref, sin_ref, ln1w_ref, ln1b_ref, wq_ref, bq_ref,
              wk_ref, bk_ref, wv_ref, bv_ref, q_ref, k_ref, v_ref):
    x = x_ref[...]
    h = _ln(x, ln1w_ref[...], ln1b_ref[...]).astype(jnp.bfloat16)
    q = jnp.dot(h, wq_ref[...], preferred_element_type=jnp.float32) + bq_ref[...]
    k = jnp.dot(h, wk_ref[...], preferred_element_type=jnp.float32) + bk_ref[...]
    v = jnp.dot(h, wv_ref[...], preferred_element_type=jnp.float32) + bv_ref[...]
    cos = cos_ref[...]
    sin = sin_ref[...]
    cq = jnp.concatenate([cos] * H, axis=1)
    sq = jnp.concatenate([sin] * H, axis=1)
    ck = jnp.concatenate([cos] * KVH, axis=1)
    sk = jnp.concatenate([sin] * KVH, axis=1)
    q1, q2 = q[:, :H * HHD], q[:, H * HHD:]
    k1, k2 = k[:, :KVH * HHD], k[:, KVH * HHD:]
    q_ref[...] = jnp.concatenate(
        [q1 * cq - q2 * sq, q2 * cq + q1 * sq], axis=1).astype(jnp.bfloat16)
    k_ref[...] = jnp.concatenate(
        [k1 * ck - k2 * sk, k2 * ck + k1 * sk], axis=1).astype(jnp.bfloat16)
    v_ref[...] = v.astype(jnp.bfloat16)


def _attn_body(q_ref, k_ref, v_ref, o_ref):
    qi = pl.program_id(0)
    q = q_ref[...]
    k = k_ref[...]
    v = v_ref[...]
    rows = qi * RB + jax.lax.broadcasted_iota(jnp.int32, (RB, S), 0)
    cols = jax.lax.broadcasted_iota(jnp.int32, (RB, S), 1)
    causal = cols <= rows
    for h in range(H):
        kv = h // (H // KVH)
        q1 = q[:, h * HHD:(h + 1) * HHD]
        q2 = q[:, H * HHD + h * HHD: H * HHD + (h + 1) * HHD]
        k1 = k[:, kv * HHD:(kv + 1) * HHD]
        k2 = k[:, KVH * HHD + kv * HHD: KVH * HHD + (kv + 1) * HHD]
        s = jax.lax.dot_general(q1, k1, (((1,), (1,)), ((), ())),
                                preferred_element_type=jnp.float32)
        s = s + jax.lax.dot_general(q2, k2, (((1,), (1,)), ((), ())),
                                    preferred_element_type=jnp.float32)
        s = jnp.where(causal, s * 0.125, jnp.float32(-1e30))
        m = jnp.max(s, axis=1, keepdims=True)
        p = jnp.exp(s - m)
        l = jnp.sum(p, axis=1, keepdims=True)
        vh = v[:, kv * HD:(kv + 1) * HD]
        o = jnp.dot(p.astype(jnp.bfloat16), vh,
                    preferred_element_type=jnp.float32) / l
        o_ref[:, h * HD:(h + 1) * HD] = o.astype(jnp.bfloat16)


def _post_body(x_ref, attn_ref, wo_ref, bo_ref, ln2w_ref, ln2b_ref, wr_ref,
               x2_ref, h2_ref, i1_ref, i2_ref, g1_ref, g2_ref):
    x2 = x_ref[...] + jnp.dot(attn_ref[...], wo_ref[...],
                              preferred_element_type=jnp.float32) + bo_ref[...]
    h2 = _ln(x2, ln2w_ref[...], ln2b_ref[...])
    logits128 = jnp.dot(h2, wr_ref[...], preferred_element_type=jnp.float32)
    logits = logits128[:, :E]
    m = jnp.max(logits, axis=1, keepdims=True)
    p = jnp.exp(logits - m)
    rp = p / jnp.sum(p, axis=1, keepdims=True)
    iota = jax.lax.broadcasted_iota(jnp.int32, (RB, E), 1)
    v1 = jnp.max(rp, axis=1, keepdims=True)
    i1 = jnp.min(jnp.where(rp == v1, iota, E), axis=1, keepdims=True)
    rp2 = jnp.where(iota == i1, -1.0, rp)
    v2 = jnp.max(rp2, axis=1, keepdims=True)
    i2 = jnp.min(jnp.where(rp2 == v2, iota, E), axis=1, keepdims=True)
    denom = v1 + v2
    x2_ref[...] = x2
    h2_ref[...] = h2
    i1_ref[...] = i1
    i2_ref[...] = i2
    g1_ref[...] = v1 / denom
    g2_ref[...] = v2 / denom


CB = 512
NCB = A // CB


def _posmap_body(ee_ref, pos_ref, bexp_ref, carry_ref, offs_ref):
    p = pl.program_id(0)
    b = pl.program_id(1)
    iotaL = jax.lax.broadcasted_iota(jnp.int32, (CB, 128), 1)
    e_blk = ee_ref[...]
    onehot = jnp.logical_and(iotaL == e_blk, iotaL < E).astype(jnp.float32)
    r = jax.lax.broadcasted_iota(jnp.int32, (CB, CB), 0)
    cc = jax.lax.broadcasted_iota(jnp.int32, (CB, CB), 1)
    ltri = (cc <= r).astype(jnp.float32)

    @pl.when(b == 0)
    def _():
        carry_ref[...] = jnp.zeros((8, 128), jnp.float32)

    carry = carry_ref[0:1, :]
    cum = jnp.dot(ltri, onehot, preferred_element_type=jnp.float32) + carry
    carry_ref[0:1, :] = carry + jnp.sum(onehot, axis=0, keepdims=True)

    @pl.when(jnp.logical_and(p == 0, b == NCB - 1))
    def _():
        cnt = carry_ref[0:1, :]
        padded = jnp.floor((cnt + (BLK - 1)) / BLK) * BLK
        li = jax.lax.broadcasted_iota(jnp.int32, (128, 128), 0)
        lj = jax.lax.broadcasted_iota(jnp.int32, (128, 128), 1)
        strict = (li < lj).astype(jnp.float32)
        offs = jnp.dot(padded, strict, preferred_element_type=jnp.float32)
        offs_ref[0:1, :] = offs
        bsv = (jax.lax.broadcasted_iota(jnp.int32, (1, 128), 1)
               * BLK).astype(jnp.float32)
        bexpv = jnp.zeros((1, 128), jnp.float32)
        for e_ in range(E):
            oe = offs[0:1, e_:e_ + 1]
            pe = padded[0:1, e_:e_ + 1]
            inseg = jnp.logical_and(bsv >= oe, bsv < oe + pe)
            bexpv = bexpv + jnp.where(inseg, float(e_), 0.0)
        total = offs[0:1, E:E + 1]
        bexpv = jnp.where(bsv >= total, float(E - 1), bexpv)
        bexp_ref[...] = bexpv.astype(jnp.int32)

    @pl.when(p == 1)
    def _():
        offs = offs_ref[0:1, :]
        posf = jnp.sum(onehot * (offs + cum - 1.0), axis=1, keepdims=True)
        pos_ref[...] = posf.astype(jnp.int32)


PB2 = A // NW


def _scatter_h2_body(h2_hbm, pos_hbm, h2s_hbm, idx_v, rows_v, sem):
    c = lax.axis_index("c")
    s = lax.axis_index("s")
    w = s * 2 + c
    base = w * PB2
    srcbase = base - S * (w // (NW // 2))
    for j in range(PB2 // CH):
        sl = pl.ds(base + j * CH, CH)
        pltpu.sync_copy(pos_hbm.at[sl], idx_v)
        pltpu.sync_copy(
            h2_hbm.at[pl.ds(pl.multiple_of(srcbase + j * CH, CH), CH)], rows_v)
        pltpu.async_copy(rows_v, h2s_hbm.at[idx_v], sem).wait()


def _gatherback_body(y_hbm, pos_hbm, buf_hbm, idx_v, rows_v, sem):
    c = lax.axis_index("c")
    s = lax.axis_index("s")
    w = s * 2 + c
    base = w * PB2
    for j in range(PB2 // CH):
        sl = pl.ds(base + j * CH, CH)
        pltpu.sync_copy(pos_hbm.at[sl], idx_v)
        pltpu.async_copy(y_hbm.at[idx_v], rows_v, sem).wait()
        pltpu.sync_copy(rows_v, buf_hbm.at[sl])


def _gffn_body(bexp_ref, h2s_ref, w1_ref, w3_ref, w2_ref, out_ref):
    h2 = h2s_ref[...].astype(jnp.bfloat16)
    a = jnp.dot(h2, w1_ref[0], preferred_element_type=jnp.float32)
    b = jnp.dot(h2, w3_ref[0], preferred_element_type=jnp.float32)
    act = (a * jax.nn.sigmoid(a) * b).astype(jnp.bfloat16)
    y = jnp.dot(act, w2_ref[0], preferred_element_type=jnp.float32)
    out_ref[...] = y


def _combine_body(x2_ref, g1_ref, g2_ref, ba_ref, bb_ref, o_ref):
    o_ref[...] = (x2_ref[...] + g1_ref[...] * ba_ref[...]
                  + g2_ref[...] * bb_ref[...])


def _split_halves(w):
    g = w.shape[-1] // HD
    w3d = w.reshape(*w.shape[:-1], g, HD)
    return jnp.concatenate(
        [w3d[..., :HHD].reshape(*w.shape[:-1], g * HHD),
         w3d[..., HHD:].reshape(*w.shape[:-1], g * HHD)], axis=-1)


def kernel(hidden_states, attention_mask, position_ids, ln1_w, ln1_b, wq, bq,
           wk, bk, wv, bv, wo, bo, ln2_w, ln2_b, w_router, w1, w3, w2):
    x = hidden_states.reshape(S, D)
    pos = position_ids.reshape(S).astype(jnp.float32)
    inv = jnp.asarray(1.0 / (THETA ** (np.arange(0, HD, 2) / HD)), jnp.float32)
    ang = pos[:, None] * inv[None, :]
    cos = jnp.cos(ang)
    sin = jnp.sin(ang)

    wq_p = _split_halves(wq)
    bq_p = _split_halves(bq[None, :])[0]
    wk_p = _split_halves(wk)
    bk_p = _split_halves(bk[None, :])[0]
    wr_pad = jnp.pad(w_router, ((0, 0), (0, 128 - E)))

    row_spec = pl.BlockSpec((RB, D), lambda i: (i, 0))
    full = lambda *shape: pl.BlockSpec(shape, lambda *i: (0,) * len(shape))

    qf, kf, vf = pl.pallas_call(
        _qkv_body,
        grid=(NRB,),
        in_specs=[
            row_spec,
            pl.BlockSpec((RB, HHD), lambda i: (i, 0)),
            pl.BlockSpec((RB, HHD), lambda i: (i, 0)),
            full(D), full(D),
            full(D, H * HD), full(H * HD),
            full(D, KVH * HD), full(KVH * HD),
            full(D, KVH * HD), full(KVH * HD),
        ],
        out_specs=[
            pl.BlockSpec((RB, H * HD), lambda i: (i, 0)),
            pl.BlockSpec((RB, KVH * HD), lambda i: (i, 0)),
            pl.BlockSpec((RB, KVH * HD), lambda i: (i, 0)),
        ],
        out_shape=[
            jax.ShapeDtypeStruct((S, H * HD), jnp.bfloat16),
            jax.ShapeDtypeStruct((S, KVH * HD), jnp.bfloat16),
            jax.ShapeDtypeStruct((S, KVH * HD), jnp.bfloat16),
        ],
        interpret=_INTERPRET,
    )(x, cos, sin, ln1_w, ln1_b, wq_p.astype(jnp.bfloat16),
      bq_p, wk_p.astype(jnp.bfloat16), bk_p, wv.astype(jnp.bfloat16), bv)

    attn = pl.pallas_call(
        _attn_body,
        grid=(NRB,),
        in_specs=[
            pl.BlockSpec((RB, H * HD), lambda i: (i, 0)),
            full(S, KVH * HD),
            full(S, KVH * HD),
        ],
        out_specs=pl.BlockSpec((RB, H * HD), lambda i: (i, 0)),
        out_shape=jax.ShapeDtypeStruct((S, H * HD), jnp.bfloat16),
        interpret=_INTERPRET,
    )(qf, kf, vf)

    col_spec = pl.BlockSpec((RB, 1), lambda i: (i, 0))
    x2, h2, i1, i2, g1, g2 = pl.pallas_call(
        _post_body,
        grid=(NRB,),
        in_specs=[
            row_spec,
            pl.BlockSpec((RB, H * HD), lambda i: (i, 0)),
            full(H * HD, D), full(D),
            full(D), full(D),
            full(D, 128),
        ],
        out_specs=[row_spec, row_spec, col_spec, col_spec, col_spec, col_spec],
        out_shape=[
            jax.ShapeDtypeStruct((S, D), jnp.float32),
            jax.ShapeDtypeStruct((S, D), jnp.float32),
            jax.ShapeDtypeStruct((S, 1), jnp.int32),
            jax.ShapeDtypeStruct((S, 1), jnp.int32),
            jax.ShapeDtypeStruct((S, 1), jnp.float32),
            jax.ShapeDtypeStruct((S, 1), jnp.float32),
        ],
        interpret=_INTERPRET,
    )(x, attn, wo.astype(jnp.bfloat16), bo, ln2_w, ln2_b, wr_pad)

    ee = jnp.concatenate([i1, i2], axis=0)
    pos2d, bexp = pl.pallas_call(
        _posmap_body,
        grid=(2, NCB),
        in_specs=[pl.BlockSpec((CB, 1), lambda p, b: (b, 0))],
        out_specs=[
            pl.BlockSpec((CB, 1), lambda p, b: (b, 0)),
            pl.BlockSpec((1, 128), lambda p, b: (0, 0)),
        ],
        out_shape=[
            jax.ShapeDtypeStruct((A, 1), jnp.int32),
            jax.ShapeDtypeStruct((1, 128), jnp.int32),
        ],
        scratch_shapes=[
            pltpu.VMEM((8, 128), jnp.float32),
            pltpu.VMEM((8, 128), jnp.float32),
        ],
        interpret=_INTERPRET,
    )(ee)
    pos = pos2d.reshape(A)
    bexp1 = bexp.reshape(128)

    mesh = plsc.VectorSubcoreMesh(core_axis_name="c", subcore_axis_name="s")
    h2s = pl.kernel(
        _scatter_h2_body,
        out_type=jax.ShapeDtypeStruct((PADMAX, D), jnp.float32),
        mesh=mesh,
        scratch_types=[
            pltpu.VMEM((CH,), jnp.int32),
            pltpu.VMEM((CH, D), jnp.float32),
            pltpu.SemaphoreType.DMA,
        ],
    )(h2, pos)

    w1b = w1.astype(jnp.bfloat16)
    w3b = w3.astype(jnp.bfloat16)
    w2b = w2.astype(jnp.bfloat16)

    y_sorted = pl.pallas_call(
        _gffn_body,
        grid_spec=pltpu.PrefetchScalarGridSpec(
            num_scalar_prefetch=1,
            grid=(NBLK,),
            in_specs=[
                pl.BlockSpec((BLK, D), lambda i, bx: (i, 0)),
                pl.BlockSpec((1, D, FF), lambda i, bx: (bx[i], 0, 0)),
                pl.BlockSpec((1, D, FF), lambda i, bx: (bx[i], 0, 0)),
                pl.BlockSpec((1, FF, D), lambda i, bx: (bx[i], 0, 0)),
            ],
            out_specs=pl.BlockSpec((BLK, D), lambda i, bx: (i, 0)),
        ),
        out_shape=jax.ShapeDtypeStruct((PADMAX, D), jnp.float32),
        interpret=_INTERPRET,
    )(bexp1, h2s, w1b, w3b, w2b)

    buf = pl.kernel(
        _gatherback_body,
        out_type=jax.ShapeDtypeStruct((A, D), jnp.float32),
        mesh=mesh,
        scratch_types=[
            pltpu.VMEM((CH,), jnp.int32),
            pltpu.VMEM((CH, D), jnp.float32),
            pltpu.SemaphoreType.DMA,
        ],
    )(y_sorted, pos)

    out = pl.pallas_call(
        _combine_body,
        grid=(NRB,),
        in_specs=[
            row_spec,
            pl.BlockSpec((RB, 1), lambda i: (i, 0)),
            pl.BlockSpec((RB, 1), lambda i: (i, 0)),
            pl.BlockSpec((RB, D), lambda i: (i, 0)),
            pl.BlockSpec((RB, D), lambda i: (i + NRB, 0)),
        ],
        out_specs=row_spec,
        out_shape=jax.ShapeDtypeStruct((S, D), jnp.float32),
        interpret=_INTERPRET,
    )(x2, g1, g2, buf, buf)

    return out.reshape(1, S, D)
